# baseline (speedup 1.0000x reference)
import functools

import jax
import jax.numpy as jnp
from jax import lax
from jax.experimental import pallas as pl
from jax.experimental.pallas import tpu as pltpu
from jax.experimental.pallas import tpu_sc as plsc

D = 64
L = 16
VOCAB = 1000000
PROWS = VOCAB // 2
NBLK = VOCAB // 128
REM = VOCAB - NBLK * 128
NW = 32
NBUF1 = 5
NBUF2 = 4
SCALE = 8.0


@functools.cache
def _build(batch: int, hist: int):
    mesh = plsc.VectorSubcoreMesh(core_axis_name="c", subcore_axis_name="s")
    assert batch == NW * 128
    nper = (NBLK + NW - 1) // NW
    ngrp1 = nper // NBUF1
    assert nper == ngrp1 * NBUF1
    ngrp2 = hist // NBUF2
    assert hist == ngrp2 * NBUF2

    @functools.partial(
        pl.kernel,
        out_type=jax.ShapeDtypeStruct((PROWS, 128), jnp.float32),
        mesh=mesh,
        scratch_types=[
            pltpu.VMEM((NBUF1, D, 128), jnp.float32),
            pltpu.VMEM((NBUF1, D, 128), jnp.float32),
        ]
        + [pltpu.SemaphoreType.DMA] * (2 * NBUF1),
        compiler_params=pltpu.CompilerParams(
            use_tc_tiling_on_sc=True, needs_layout_passes=False
        ),
    )
    def stage1(tt_hbm, tail_hbm, tr_hbm, a_v, b_v, *sems):
        isem = sems[:NBUF1]
        osem = sems[NBUF1:]
        wid = lax.axis_index("s") * 2 + lax.axis_index("c")
        iota = lax.iota(jnp.int32, L)
        rowsel = [iota + 16 * (lg & 3) for lg in range(8)]

        def transpose_block(b, nq):
            def qbody(q, c2):
                for lg in range(8):
                    col = jnp.broadcast_to(2 * q + (lg >> 2), (L,)).astype(
                        jnp.int32
                    )
                    vals = plsc.load_gather(a_v.at[b], [rowsel[lg], col])
                    b_v[b, q, pl.ds(lg * L, L)] = vals * SCALE
                return c2

            lax.fori_loop(0, nq, qbody, 0, unroll=2)

        for b in range(NBUF1):
            pltpu.async_copy(
                tt_hbm.at[:, pl.ds((b * NW + wid) * 128, 128)],
                a_v.at[b],
                isem[b],
            )

        def group(jg, carry):
            for b in range(NBUF1):
                j = jg * NBUF1 + b
                blk = j * NW + wid

                @pl.when(blk < NBLK)
                def _():
                    pltpu.make_async_copy(
                        tt_hbm.at[:, pl.ds(blk * 128, 128)], a_v.at[b], isem[b]
                    ).wait()

                    @pl.when(j >= NBUF1)
                    def _():
                        pltpu.make_async_copy(
                            b_v.at[b],
                            tr_hbm.at[pl.ds((blk - NBUF1 * NW) * D, D)],
                            osem[b],
                        ).wait()

                    transpose_block(b, D)

                    nblk = (j + NBUF1) * NW + wid

                    @pl.when(nblk < NBLK)
                    def _():
                        pltpu.async_copy(
                            tt_hbm.at[:, pl.ds(nblk * 128, 128)],
                            a_v.at[b],
                            isem[b],
                        )

                    pltpu.async_copy(
                        b_v.at[b], tr_hbm.at[pl.ds(blk * D, D)], osem[b]
                    )

            return carry

        lax.fori_loop(0, ngrp1, group, 0)

        for t in range(nper - 1 - NBUF1, nper):
            b = t % NBUF1
            blk = t * NW + wid
            nblk = (t + NBUF1) * NW + wid

            @pl.when((blk < NBLK) & (nblk >= NBLK))
            def _():
                pltpu.make_async_copy(
                    b_v.at[b], tr_hbm.at[pl.ds(blk * D, D)], osem[b]
                ).wait()

        @pl.when(wid == NW - 1)
        def _():
            pltpu.sync_copy(tail_hbm, tr_hbm.at[pl.ds(NBLK * D, REM // 2)])

    @functools.partial(
        pl.kernel,
        out_type=jax.ShapeDtypeStruct((hist, D, batch), jnp.float32),
        mesh=mesh,
        scratch_types=[
            pltpu.VMEM((hist, 128), jnp.int32),
            pltpu.VMEM((NBUF2, 128), jnp.int32),
            pltpu.VMEM((NBUF2, 128, 128), jnp.float32),
            pltpu.VMEM((NBUF2, D, 128), jnp.float32),
        ]
        + [pltpu.SemaphoreType.DMA] * (2 * NBUF2),
        compiler_params=pltpu.CompilerParams(
            use_tc_tiling_on_sc=True, needs_layout_passes=False
        ),
    )
    def stage2(xt_hbm, tr_hbm, out_hbm, idx_v, pidx_v, g_v, s_v, *sems):
        gsem = sems[:NBUF2]
        ssem = sems[NBUF2:]
        wid = lax.axis_index("s") * 2 + lax.axis_index("c")
        iota = lax.iota(jnp.int32, L)
        bsel = [iota + 16 * g for g in range(8)]

        pltpu.sync_copy(xt_hbm.at[:, pl.ds(wid * 128, 128)], idx_v)

        def prep_fire(h, b):
            for g in range(8):
                iv = idx_v[h, pl.ds(g * L, L)]
                pidx_v[b, pl.ds(g * L, L)] = lax.shift_right_logical(iv, 1)
            pltpu.async_copy(tr_hbm.at[pidx_v.at[b]], g_v.at[b], gsem[b])

        for b in range(NBUF2):
            prep_fire(b, b)

        def group(hg, carry):
            for b in range(NBUF2):
                h = hg * NBUF2 + b
                pltpu.make_async_copy(
                    tr_hbm.at[pidx_v.at[b]], g_v.at[b], gsem[b]
                ).wait()

                @pl.when(h >= NBUF2)
                def _():
                    pltpu.make_async_copy(
                        s_v.at[b],
                        out_hbm.at[h - NBUF2, :, pl.ds(wid * 128, 128)],
                        ssem[b],
                    ).wait()

                cbase = []
                for g in range(8):
                    iv = idx_v[h, pl.ds(g * L, L)]
                    cbase.append(lax.shift_left(jnp.bitwise_and(iv, 1), 6))

                def dbody(d, c2):
                    for g in range(8):
                        vals = plsc.load_gather(
                            g_v.at[b], [bsel[g], cbase[g] + d]
                        )
                        s_v[b, d, pl.ds(g * L, L)] = vals
                    return c2

                lax.fori_loop(0, D, dbody, 0, unroll=2)

                @pl.when(h + NBUF2 < hist)
                def _():
                    prep_fire(h + NBUF2, b)

                pltpu.async_copy(
                    s_v.at[b], out_hbm.at[h, :, pl.ds(wid * 128, 128)], ssem[b]
                )
            return carry

        lax.fori_loop(0, ngrp2, group, 0)

        for b in range(NBUF2):
            h = hist - NBUF2 + b
            pltpu.make_async_copy(
                s_v.at[b], out_hbm.at[h, :, pl.ds(wid * 128, 128)], ssem[b]
            ).wait()

    return stage1, stage2


def kernel(x, table):
    batch, hist = x.shape
    stage1, stage2 = _build(batch, hist)
    tail = (table[NBLK * 128 :] * SCALE).reshape(REM // 2, 128)
    tr = stage1(table.T, tail)
    o3 = stage2(x.T.astype(jnp.int32), tr)
    return o3.transpose(2, 0, 1)


# speedup vs baseline: 1.1057x; 1.1057x over previous
import functools

import jax
import jax.numpy as jnp
from jax import lax
from jax.experimental import pallas as pl
from jax.experimental.pallas import tpu as pltpu
from jax.experimental.pallas import tpu_sc as plsc

D = 64
L = 16
VOCAB = 1000000
PROWS = VOCAB // 2
NBLK = VOCAB // 128
REM = VOCAB - NBLK * 128
NW = 32
NBUF1 = 5
NBUF2 = 4
SCALE = 8.0


@functools.cache
def _build(batch: int, hist: int):
    mesh = plsc.VectorSubcoreMesh(core_axis_name="c", subcore_axis_name="s")
    assert batch == NW * 128
    nper = (NBLK + NW - 1) // NW
    ngrp1 = nper // NBUF1
    assert nper == ngrp1 * NBUF1
    ngrp2 = hist // NBUF2
    assert hist == ngrp2 * NBUF2

    @functools.partial(
        pl.kernel,
        out_type=jax.ShapeDtypeStruct((PROWS, 128), jnp.float32),
        mesh=mesh,
        scratch_types=[
            pltpu.VMEM((NBUF1, D, 128), jnp.float32),
            pltpu.VMEM((NBUF1, D, 128), jnp.float32),
        ]
        + [pltpu.SemaphoreType.DMA] * (2 * NBUF1),
        compiler_params=pltpu.CompilerParams(
            use_tc_tiling_on_sc=True, needs_layout_passes=False
        ),
    )
    def stage1(tt_hbm, tail_hbm, tr_hbm, a_v, b_v, *sems):
        isem = sems[:NBUF1]
        osem = sems[NBUF1:]
        wid = lax.axis_index("s") * 2 + lax.axis_index("c")
        iota = lax.iota(jnp.int32, L)
        rowsel = [iota + 16 * (lg & 3) for lg in range(8)]

        def transpose_block(b, nq):
            def dbody(d, c2):
                for jg in range(8):
                    vals = a_v[b, d, pl.ds(jg * L, L)] * SCALE
                    j = iota + jg * L
                    rows = lax.shift_right_logical(j, 1)
                    cols = lax.shift_left(jnp.bitwise_and(j, 1), 6) + d
                    plsc.store_scatter(b_v.at[b], [rows, cols], vals)
                return c2

            lax.fori_loop(0, nq, dbody, 0, unroll=2)

        for b in range(NBUF1):
            pltpu.async_copy(
                tt_hbm.at[:, pl.ds((b * NW + wid) * 128, 128)],
                a_v.at[b],
                isem[b],
            )

        def group(jg, carry):
            for b in range(NBUF1):
                j = jg * NBUF1 + b
                blk = j * NW + wid

                @pl.when(blk < NBLK)
                def _():
                    pltpu.make_async_copy(
                        tt_hbm.at[:, pl.ds(blk * 128, 128)], a_v.at[b], isem[b]
                    ).wait()

                    @pl.when(j >= NBUF1)
                    def _():
                        pltpu.make_async_copy(
                            b_v.at[b],
                            tr_hbm.at[pl.ds((blk - NBUF1 * NW) * D, D)],
                            osem[b],
                        ).wait()

                    transpose_block(b, D)

                    nblk = (j + NBUF1) * NW + wid

                    @pl.when(nblk < NBLK)
                    def _():
                        pltpu.async_copy(
                            tt_hbm.at[:, pl.ds(nblk * 128, 128)],
                            a_v.at[b],
                            isem[b],
                        )

                    pltpu.async_copy(
                        b_v.at[b], tr_hbm.at[pl.ds(blk * D, D)], osem[b]
                    )

            return carry

        lax.fori_loop(0, ngrp1, group, 0)

        for t in range(nper - 1 - NBUF1, nper):
            b = t % NBUF1
            blk = t * NW + wid
            nblk = (t + NBUF1) * NW + wid

            @pl.when((blk < NBLK) & (nblk >= NBLK))
            def _():
                pltpu.make_async_copy(
                    b_v.at[b], tr_hbm.at[pl.ds(blk * D, D)], osem[b]
                ).wait()

        @pl.when(wid == NW - 1)
        def _():
            pltpu.sync_copy(tail_hbm, tr_hbm.at[pl.ds(NBLK * D, REM // 2)])

    @functools.partial(
        pl.kernel,
        out_type=jax.ShapeDtypeStruct((hist, D, batch), jnp.float32),
        mesh=mesh,
        scratch_types=[
            pltpu.VMEM((hist, 128), jnp.int32),
            pltpu.VMEM((NBUF2, 128), jnp.int32),
            pltpu.VMEM((NBUF2, 128, 128), jnp.float32),
            pltpu.VMEM((NBUF2, D, 128), jnp.float32),
        ]
        + [pltpu.SemaphoreType.DMA] * (2 * NBUF2),
        compiler_params=pltpu.CompilerParams(
            use_tc_tiling_on_sc=True, needs_layout_passes=False
        ),
    )
    def stage2(xt_hbm, tr_hbm, out_hbm, idx_v, pidx_v, g_v, s_v, *sems):
        gsem = sems[:NBUF2]
        ssem = sems[NBUF2:]
        wid = lax.axis_index("s") * 2 + lax.axis_index("c")
        iota = lax.iota(jnp.int32, L)
        bsel = [iota + 16 * g for g in range(8)]

        pltpu.sync_copy(xt_hbm.at[:, pl.ds(wid * 128, 128)], idx_v)

        def prep_fire(h, b):
            for g in range(8):
                iv = idx_v[h, pl.ds(g * L, L)]
                pidx_v[b, pl.ds(g * L, L)] = lax.shift_right_logical(iv, 1)
            pltpu.async_copy(tr_hbm.at[pidx_v.at[b]], g_v.at[b], gsem[b])

        for b in range(NBUF2):
            prep_fire(b, b)

        def group(hg, carry):
            for b in range(NBUF2):
                h = hg * NBUF2 + b
                pltpu.make_async_copy(
                    tr_hbm.at[pidx_v.at[b]], g_v.at[b], gsem[b]
                ).wait()

                @pl.when(h >= NBUF2)
                def _():
                    pltpu.make_async_copy(
                        s_v.at[b],
                        out_hbm.at[h - NBUF2, :, pl.ds(wid * 128, 128)],
                        ssem[b],
                    ).wait()

                cbase = []
                for g in range(8):
                    iv = idx_v[h, pl.ds(g * L, L)]
                    cbase.append(lax.shift_left(jnp.bitwise_and(iv, 1), 6))

                def dbody(d, c2):
                    for g in range(8):
                        vals = plsc.load_gather(
                            g_v.at[b], [bsel[g], cbase[g] + d]
                        )
                        s_v[b, d, pl.ds(g * L, L)] = vals
                    return c2

                lax.fori_loop(0, D, dbody, 0, unroll=2)

                @pl.when(h + NBUF2 < hist)
                def _():
                    prep_fire(h + NBUF2, b)

                pltpu.async_copy(
                    s_v.at[b], out_hbm.at[h, :, pl.ds(wid * 128, 128)], ssem[b]
                )
            return carry

        lax.fori_loop(0, ngrp2, group, 0)

        for b in range(NBUF2):
            h = hist - NBUF2 + b
            pltpu.make_async_copy(
                s_v.at[b], out_hbm.at[h, :, pl.ds(wid * 128, 128)], ssem[b]
            ).wait()

    return stage1, stage2


def kernel(x, table):
    batch, hist = x.shape
    stage1, stage2 = _build(batch, hist)
    tail = (table[NBLK * 128 :] * SCALE).reshape(REM // 2, 128)
    tr = stage1(table.T, tail)
    o3 = stage2(x.T.astype(jnp.int32), tr)
    return o3.transpose(2, 0, 1)
